# baseline (speedup 1.0000x reference)
import jax
import jax.numpy as jnp
from jax import lax
from jax.experimental import pallas as pl
from jax.experimental.pallas import tpu as pltpu, tpu_sc as plsc

B, S, D = 4, 2048, 1024
NC, NS, L = 2, 16, 16
NW = NC * NS
RPW = (B * S) // NW
CHR = 16
NCH = RPW // CHR
VPR = D // L


def _sc_body(x_hbm, p_hbm, o_hbm, in_buf, out_buf, p_v):
    c = lax.axis_index("c")
    s = lax.axis_index("s")
    wid = s * NC + c
    base = wid * RPW
    pbase = lax.rem(base, S)
    pltpu.sync_copy(p_hbm.at[pl.ds(pbase, RPW)], p_v)

    def chunk(ci, carry):
        flat0 = (base + ci * CHR) * D
        pltpu.sync_copy(x_hbm.at[pl.ds(flat0, CHR * D)], in_buf)

        def row(ri, carry2):
            pb = plsc.load_gather(
                p_v, [jnp.full((L,), ci * CHR + ri, jnp.int32)]
            )

            def vec(vi, carry3):
                off = ri * D + vi * L
                out_buf[pl.ds(off, L)] = in_buf[pl.ds(off, L)] + pb
                return 0

            return lax.fori_loop(0, VPR, vec, 0)

        lax.fori_loop(0, CHR, row, 0)
        pltpu.sync_copy(out_buf, o_hbm.at[pl.ds(flat0, CHR * D)])
        return 0

    lax.fori_loop(0, NCH, chunk, 0)


def kernel(inputs, pos_weight):
    x1 = inputs.reshape(B * S * D)
    p1 = pos_weight.reshape(S)
    sc = pl.kernel(
        _sc_body,
        out_type=jax.ShapeDtypeStruct((B * S * D,), jnp.float32),
        mesh=plsc.VectorSubcoreMesh(core_axis_name="c", subcore_axis_name="s"),
        scratch_types=[
            pltpu.VMEM((CHR * D,), jnp.float32),
            pltpu.VMEM((CHR * D,), jnp.float32),
            pltpu.VMEM((RPW,), jnp.float32),
        ],
        compiler_params=pltpu.CompilerParams(needs_layout_passes=False),
    )
    return sc(x1, p1).reshape(B, S, D)


# speedup vs baseline: 1.2338x; 1.2338x over previous
import jax
import jax.numpy as jnp
from jax import lax
from jax.experimental import pallas as pl
from jax.experimental.pallas import tpu as pltpu, tpu_sc as plsc

B, S, D = 4, 2048, 1024
NC, NS, L = 2, 16, 16
NW = NC * NS
RPW = (B * S) // NW
CHR = 16
NCH = RPW // CHR
VPR = D // L
UNROLL = 8


def _sc_body(x_hbm, p_hbm, o_hbm,
             in0, in1, out0, out1, p_v,
             isem0, isem1, osem0, osem1):
    c = lax.axis_index("c")
    s = lax.axis_index("s")
    wid = s * NC + c
    base = wid * RPW
    pltpu.sync_copy(p_hbm.at[pl.ds(lax.rem(base, S), RPW)], p_v)

    def in_copy(ci, buf, sem):
        flat0 = (base + ci * CHR) * D
        return pltpu.make_async_copy(x_hbm.at[pl.ds(flat0, CHR * D)], buf, sem)

    def out_copy(ci, buf, sem):
        flat0 = (base + ci * CHR) * D
        return pltpu.make_async_copy(buf, o_hbm.at[pl.ds(flat0, CHR * D)], sem)

    def compute(ci, ibuf, obuf):
        def row(ri, _):
            pb = plsc.load_gather(
                p_v, [jnp.full((L,), ci * CHR + ri, jnp.int32)]
            )

            def vec(vi, _):
                for u in range(UNROLL):
                    off = ri * D + (vi * UNROLL + u) * L
                    obuf[pl.ds(off, L)] = ibuf[pl.ds(off, L)] + pb
                return 0

            return lax.fori_loop(0, VPR // UNROLL, vec, 0)

        lax.fori_loop(0, CHR, row, 0)

    bufs = ((in0, isem0, out0, osem0), (in1, isem1, out1, osem1))

    in_copy(0, in0, isem0).start()
    in_copy(1, in1, isem1).start()

    def step(j, _):
        for b in range(2):
            ibuf, isem, obuf, osem = bufs[b]
            ci = 2 * j + b
            in_copy(ci, ibuf, isem).wait()

            @pl.when(j > 0)
            def _():
                out_copy(jnp.maximum(ci - 2, 0), obuf, osem).wait()

            compute(ci, ibuf, obuf)
            out_copy(ci, obuf, osem).start()

            @pl.when(ci + 2 < NCH)
            def _():
                in_copy(ci + 2, ibuf, isem).start()
        return 0

    lax.fori_loop(0, NCH // 2, step, 0)
    out_copy(NCH - 2, out0, osem0).wait()
    out_copy(NCH - 1, out1, osem1).wait()


def kernel(inputs, pos_weight):
    x1 = inputs.reshape(B * S * D)
    p1 = pos_weight.reshape(S)
    sc = pl.kernel(
        _sc_body,
        out_type=jax.ShapeDtypeStruct((B * S * D,), jnp.float32),
        mesh=plsc.VectorSubcoreMesh(core_axis_name="c", subcore_axis_name="s"),
        scratch_types=[
            pltpu.VMEM((CHR * D,), jnp.float32),
            pltpu.VMEM((CHR * D,), jnp.float32),
            pltpu.VMEM((CHR * D,), jnp.float32),
            pltpu.VMEM((CHR * D,), jnp.float32),
            pltpu.VMEM((RPW,), jnp.float32),
            pltpu.SemaphoreType.DMA,
            pltpu.SemaphoreType.DMA,
            pltpu.SemaphoreType.DMA,
            pltpu.SemaphoreType.DMA,
        ],
        compiler_params=pltpu.CompilerParams(needs_layout_passes=False),
    )
    return sc(x1, p1).reshape(B, S, D)


# speedup vs baseline: 6.6539x; 5.3929x over previous
import jax
import jax.numpy as jnp
from jax import lax
from jax.experimental import pallas as pl
from jax.experimental.pallas import tpu as pltpu

B, S, D = 4, 2048, 1024
R_BLK = 1024
G = (B * S) // R_BLK


def _body(a_ref, b_ref, p_ref, o_ref):
    g = pl.program_id(0)

    @pl.when(lax.rem(g, 2) == 0)
    def _():
        o_ref[...] = a_ref[...] + p_ref[...]

    @pl.when(lax.rem(g, 2) == 1)
    def _():
        o_ref[...] = b_ref[...] + p_ref[...]


def kernel(inputs, pos_weight):
    x2 = inputs.reshape(B * S, D)
    p2 = jnp.tile(pos_weight, (B, 1))
    out = pl.pallas_call(
        _body,
        grid=(G,),
        in_specs=[
            pl.BlockSpec((R_BLK, D), lambda g: ((g // 2) * 2, 0)),
            pl.BlockSpec((R_BLK, D), lambda g: ((g // 2) * 2 + 1, 0)),
            pl.BlockSpec((R_BLK, 1), lambda g: (g, 0)),
        ],
        out_specs=pl.BlockSpec((R_BLK, D), lambda g: (g, 0)),
        out_shape=jax.ShapeDtypeStruct((B * S, D), jnp.float32),
        compiler_params=pltpu.CompilerParams(
            vmem_limit_bytes=100 * 1024 * 1024,
        ),
    )(x2, x2, p2)
    return out.reshape(B, S, D)


# speedup vs baseline: 7.0501x; 1.0595x over previous
import jax
import jax.numpy as jnp
from jax import lax
from jax.experimental import pallas as pl
from jax.experimental.pallas import tpu as pltpu
from jax._src.pallas.core import RevisitMode

B, S, D = 4, 2048, 1024
H = S // 2


def _add_body(x_ref, p_ref, o_ref):
    g = pl.program_id(0)
    h = lax.rem(g, 2)
    o_ref[...] = (
        x_ref[:, pl.ds(h * H, H), :] + p_ref[pl.ds(h * H, H), :][None, :, :]
    )


def kernel(inputs, pos_weight):
    return pl.pallas_call(
        _add_body,
        grid=(2 * B,),
        in_specs=[
            pl.BlockSpec((1, S, D), lambda g: (g // 2, 0, 0),
                         pipeline_mode=pl.Buffered(
                             buffer_count=2, revisit=RevisitMode.IMMEDIATE)),
            pl.BlockSpec((S, 1), lambda g: (0, 0)),
        ],
        out_specs=pl.BlockSpec((1, H, D), lambda g: (g // 2, g % 2, 0)),
        out_shape=jax.ShapeDtypeStruct((B, S, D), jnp.float32),
        compiler_params=pltpu.CompilerParams(
            vmem_limit_bytes=100 * 1024 * 1024,
        ),
    )(inputs, pos_weight)


# speedup vs baseline: 9.0542x; 1.2843x over previous
import jax
import jax.numpy as jnp
from jax.experimental import pallas as pl
from jax.experimental.pallas import tpu as pltpu

B, S, D = 4, 2048, 1024
S_BLK = 1024


def _add_body(x_ref, p_ref, o_ref):
    o_ref[...] = x_ref[...] + p_ref[...][None, :, :]


def kernel(inputs, pos_weight):
    return pl.pallas_call(
        _add_body,
        grid=(B,),
        in_specs=[
            pl.BlockSpec((1, S, D), lambda b: (b, 0, 0)),
            pl.BlockSpec((S, 1), lambda b: (0, 0)),
        ],
        out_specs=pl.BlockSpec((1, S, D), lambda b: (b, 0, 0)),
        out_shape=jax.ShapeDtypeStruct((B, S, D), jnp.float32),
        compiler_params=pltpu.CompilerParams(
            vmem_limit_bytes=100 * 1024 * 1024,
        ),
    )(inputs, pos_weight)
